# baseline (speedup 1.0000x reference)
import functools

import jax
import jax.numpy as jnp
from jax import lax
from jax.experimental import pallas as pl
from jax.experimental.pallas import tpu as pltpu
from jax.experimental.pallas import tpu_sc as plsc

BLK = 2000


def _bd4(w):
    return jnp.kron(jnp.eye(4, dtype=jnp.float32), w)


def _t4(b):
    return jnp.tile(b, 4)


def _mm_kernel(x_ref, w0, w1, w2, o0, o1, o2):
    xb = x_ref[...]
    for w, o in ((w0, o0), (w1, o1), (w2, o2)):
        o[...] = jnp.dot(xb, w[...], preferred_element_type=jnp.float32)


def _premul3(x4, bdw1s):
    n4, d4 = x4.shape
    blk4 = n4
    grid = 1
    outs = [jax.ShapeDtypeStruct((n4, 128), jnp.float32)] * 3
    return pl.pallas_call(
        _mm_kernel,
        grid=(grid,),
        in_specs=[pl.BlockSpec((blk4, d4), lambda i: (i, 0))] + [
            pl.BlockSpec((d4, 128), lambda i: (0, 0))] * 3,
        out_specs=[pl.BlockSpec((blk4, 128), lambda i: (i, 0))] * 3,
        out_shape=outs,
    )(x4, *bdw1s)


def _segsum3(y0, y1, y2, e0, e1, e2):
    n = y0.shape[0]
    e = e0.shape[1]
    info = plsc.get_sparse_core_info()
    nc, ns = info.num_cores, info.num_subcores
    nw = nc * ns
    epw = e // nw
    ch = 400
    nch = epw // ch
    nb = 4
    assert ch * nch == epw and epw * nw == e
    rpt = (-(-n // ns) + 7) // 8 * 8
    n_pad = rpt * ns

    @functools.partial(
        pl.kernel,
        out_type=[jax.ShapeDtypeStruct((n_pad, 32), jnp.float32)] * 6,
        mesh=plsc.VectorSubcoreMesh(core_axis_name="c", subcore_axis_name="s"),
        scratch_types=[
            pltpu.VMEM_SHARED((n_pad, 32), jnp.float32),
            pltpu.VMEM_SHARED((n_pad, 32), jnp.float32),
            pltpu.VMEM_SHARED((n_pad, 32), jnp.float32),
            [pltpu.VMEM((ch,), jnp.int32) for _ in range(nb)],
            [pltpu.VMEM((ch,), jnp.int32) for _ in range(nb)],
            [pltpu.VMEM((ch, 32), jnp.float32) for _ in range(nb)],
            [pltpu.SemaphoreType.DMA for _ in range(nb)],
            [pltpu.SemaphoreType.DMA for _ in range(nb)],
            [pltpu.SemaphoreType.DMA for _ in range(nb)],
        ],
        compiler_params=pltpu.CompilerParams(use_tc_tiling_on_sc=False),
    )
    def k(y0h, y1h, y2h, e0h, e1h, e2h,
          o00, o01, o10, o11, o20, o21,
          a0, a1, a2, sidx, didx, rows, isem, gsem, ssem):
        cid = lax.axis_index("c")
        sid = lax.axis_index("s")
        wid = sid * nc + cid
        base = wid * epw
        rows_a = rows[0]

        def prefetch(eh, c, b):
            pltpu.async_copy(eh.at[0, pl.ds(base + c * ch, ch)],
                             sidx[b], isem[b])
            pltpu.async_copy(eh.at[1, pl.ds(base + c * ch, ch)],
                             didx[b], isem[b])

        def wait_prefetch(eh, c, b):
            pltpu.make_async_copy(eh.at[0, pl.ds(base + c * ch, ch)],
                                  sidx[b], isem[b]).wait()
            pltpu.make_async_copy(eh.at[1, pl.ds(base + c * ch, ch)],
                                  didx[b], isem[b]).wait()

        for b in range(nb):
            prefetch(e0h, b, b)

        zero16 = jnp.zeros((16,), jnp.float32)

        def zb(i, carry):
            rows_a[i, pl.ds(0, 16)] = zero16
            rows_a[i, pl.ds(16, 16)] = zero16
            return carry

        lax.fori_loop(0, ch, zb, 0)
        r0 = sid * rpt
        nfull, rem = divmod(rpt, ch)
        zdscs = []
        zi = 0
        for a in (a0, a1, a2):
            for j in range(nfull):
                zdscs.append(pltpu.async_copy(
                    rows_a, a.at[pl.ds(r0 + j * ch, ch)], gsem[zi % nb]))
                zi += 1
            if rem:
                zdscs.append(pltpu.async_copy(
                    rows_a.at[pl.ds(0, rem)],
                    a.at[pl.ds(r0 + nfull * ch, rem)], gsem[zi % nb]))
                zi += 1
        for dsc in zdscs:
            dsc.wait()
        plsc.subcore_barrier()

        ngrp = nch // nb
        npeel = nch - ngrp * nb
        sets = ((y0h, e0h, a0), (y1h, e1h, a1), (y2h, e2h, a2))
        for ksi, (yh, eh, a) in enumerate(sets):
            if ksi > 0:
                for b in range(nb):
                    prefetch(eh, b, b)

            def grp(g, carry):
                for b in range(nb):
                    c = g * nb + b
                    wait_prefetch(eh, c, b)
                    pltpu.async_copy(yh.at[sidx[b]], rows[b], gsem[b])
                for b in range(nb):
                    pltpu.make_async_copy(yh.at[sidx[b]], rows[b],
                                          gsem[b]).wait()
                    pltpu.async_copy(rows[b], a.at[didx[b]], ssem[b],
                                     add=True)
                for b in range(nb):
                    c_next = g * nb + b + nb

                    @pl.when(c_next < nch)
                    def _():
                        pltpu.make_async_copy(rows[b], a.at[didx[b]],
                                              ssem[b]).wait()
                        prefetch(eh, c_next, b)
                return carry

            lax.fori_loop(0, ngrp, grp, 0)
            for b in range(npeel):
                c = ngrp * nb + b
                wait_prefetch(eh, c, b)
                pltpu.async_copy(yh.at[sidx[b]], rows[b], gsem[b])
            for b in range(npeel):
                pltpu.make_async_copy(yh.at[sidx[b]], rows[b], gsem[b]).wait()
                pltpu.async_copy(rows[b], a.at[didx[b]], ssem[b], add=True)
            for b in range(nb):
                pltpu.make_async_copy(rows[b], a.at[didx[b]], ssem[b]).wait()
        plsc.subcore_barrier()

        for a, oc0, oc1 in ((a0, o00, o01), (a1, o10, o11), (a2, o20, o21)):
            @pl.when(cid == 0)
            def _():
                pltpu.sync_copy(a.at[pl.ds(r0, rpt)], oc0.at[pl.ds(r0, rpt)])

            @pl.when(cid == 1)
            def _():
                pltpu.sync_copy(a.at[pl.ds(r0, rpt)], oc1.at[pl.ds(r0, rpt)])

    return k(y0, y1, y2, e0, e1, e2)


def _mid_kernel(y0, y1, y2, p00, p01, p10, p11, p20, p21, b1s, w2s, b2s,
                m1w1s, m1b1, m1w2, m1b2, w1s2, o0, o1, o2):
    upre = m1b1[...]
    for i, (y, pa, pb) in enumerate(((y0, p00, p01), (y1, p10, p11),
                                     (y2, p20, p21))):
        nr = y.shape[0]
        pre = (y[...] + pa[pl.ds(0, nr), :] + pb[pl.ds(0, nr), :]
               + b1s[pl.ds(i, 1)])
        t = jnp.dot(jnp.maximum(pre, 0.0), w2s[i],
                    preferred_element_type=jnp.float32) + b2s[pl.ds(i, 1)]
        upre = upre + jnp.dot(t, m1w1s[i], preferred_element_type=jnp.float32)
    u = jnp.dot(jnp.maximum(upre, 0.0), m1w2[...],
                preferred_element_type=jnp.float32) + m1b2[...]
    for i, o in enumerate((o0, o1, o2)):
        o[...] = jnp.dot(u, w1s2[i], preferred_element_type=jnp.float32)


def _mid(y0, y1, y2, ps, b1s, w2s, b2s, m1w1s, m1b1, m1w2, m1b2, w1s2):
    n4 = y0.shape[0]
    blk4 = n4
    grid = 1
    yspec = pl.BlockSpec((blk4, 128), lambda i: (i, 0))
    pspec = pl.BlockSpec(ps[0].shape, lambda i: (0, 0))
    full = lambda s: pl.BlockSpec(s, lambda i: tuple(0 for _ in s))
    return pl.pallas_call(
        _mid_kernel,
        grid=(grid,),
        in_specs=[yspec] * 3 + [pspec] * 6 + [
            full((3, 128)), full((3, 128, 128)), full((3, 128)),
            full((3, 128, 128)), full((1, 128)), full((128, 128)),
            full((1, 128)), full((3, 128, 128)),
        ],
        out_specs=[yspec] * 3,
        out_shape=[jax.ShapeDtypeStruct((n4, 128), jnp.float32)] * 3,
    )(y0, y1, y2, *ps, b1s, w2s, b2s, m1w1s, m1b1, m1w2, m1b2, w1s2)


def _fin_kernel(z0, z1, z2, q00, q01, q10, q11, q20, q21, batch_ref,
                b1s, w2s, b2s, m2w1s, m2b1, m2w2, m2b2, linw, linb,
                out_ref, acc):
    i = pl.program_id(0)
    nblk = pl.num_programs(0)
    vpre = m2b1[...]
    for k, (z, qa, qb) in enumerate(((z0, q00, q01), (z1, q10, q11),
                                     (z2, q20, q21))):
        nr = z.shape[0]
        pre = (z[...] + qa[pl.ds(0, nr), :] + qb[pl.ds(0, nr), :]
               + b1s[pl.ds(k, 1)])
        t = jnp.dot(jnp.maximum(pre, 0.0), w2s[k],
                    preferred_element_type=jnp.float32) + b2s[pl.ds(k, 1)]
        vpre = vpre + jnp.dot(jnp.maximum(t, 0.0), m2w1s[k],
                              preferred_element_type=jnp.float32)
    v = jnp.dot(jnp.maximum(vpre, 0.0), m2w2[...],
                preferred_element_type=jnp.float32) + m2b2[...]
    g = acc.shape[0]
    blk4 = v.shape[0]
    part = jnp.zeros((g, 32), jnp.float32)
    for j in range(4):
        bj = batch_ref[:, pl.ds(j, 1)]
        ohjt = jnp.where(
            jax.lax.broadcasted_iota(jnp.int32, (blk4, g), 1) == bj, 1.0, 0.0)
        vj = v[:, 32 * j:32 * j + 32]
        part = part + jax.lax.dot_general(
            ohjt, vj, (((0,), (0,)), ((), ())),
            preferred_element_type=jnp.float32)

    @pl.when(i == 0)
    def _():
        acc[...] = jnp.zeros_like(acc)

    acc[...] += part

    @pl.when(i == nblk - 1)
    def _():
        out_ref[...] = jnp.dot(acc[...], linw[...],
                               preferred_element_type=jnp.float32) + linb[...]


def _final(z0, z1, z2, qs, batch4, b1s, w2s, b2s,
           m2w1s, m2b1, m2w2, m2b2, linw, linb, g):
    n4 = z0.shape[0]
    blk4 = n4
    grid = 1
    zspec = pl.BlockSpec((blk4, 128), lambda i: (i, 0))
    qspec = pl.BlockSpec(qs[0].shape, lambda i: (0, 0))
    full = lambda s: pl.BlockSpec(s, lambda i: tuple(0 for _ in s))
    return pl.pallas_call(
        _fin_kernel,
        grid=(grid,),
        in_specs=[zspec] * 3 + [qspec] * 6 + [
            pl.BlockSpec((blk4, 4), lambda i: (i, 0)),
            full((3, 128)), full((3, 128, 128)), full((3, 128)),
            full((3, 128, 128)), full((1, 128)), full((128, 128)),
            full((1, 128)), full((32, 1)), full((1, 1)),
        ],
        out_specs=full((g, 1)),
        out_shape=jax.ShapeDtypeStruct((g, 1), jnp.float32),
        scratch_shapes=[pltpu.VMEM((g, 32), jnp.float32)],
    )(z0, z1, z2, *qs, batch4, b1s, w2s, b2s,
      m2w1s, m2b1, m2w2, m2b2, linw, linb)


def kernel(x, edge_index_0, edge_index_1, edge_index_2, batch,
           c11_W1, c11_b1, c11_W2, c11_b2,
           c12_W1, c12_b1, c12_W2, c12_b2,
           c13_W1, c13_b1, c13_W2, c13_b2,
           c21_W1, c21_b1, c21_W2, c21_b2,
           c22_W1, c22_b1, c22_W2, c22_b2,
           c23_W1, c23_b1, c23_W2, c23_b2,
           mlp1_W1, mlp1_b1, mlp1_W2, mlp1_b2,
           mlp2_W1, mlp2_b1, mlp2_W2, mlp2_b2,
           lin_W, lin_b):
    n = x.shape[0]
    g = 64

    bdw1s = [jnp.kron(jnp.eye(4, dtype=jnp.float32), w)
             for w in (c11_W1, c12_W1, c13_W1)]
    b1s_1 = jnp.stack([_t4(c11_b1), _t4(c12_b1), _t4(c13_b1)])
    w2s_1 = jnp.stack([_bd4(c11_W2), _bd4(c12_W2), _bd4(c13_W2)])
    b2s_1 = jnp.stack([_t4(c11_b2), _t4(c12_b2), _t4(c13_b2)])
    b1s_2 = jnp.stack([_t4(c21_b1), _t4(c22_b1), _t4(c23_b1)])
    w2s_2 = jnp.stack([_bd4(c21_W2), _bd4(c22_W2), _bd4(c23_W2)])
    b2s_2 = jnp.stack([_t4(c21_b2), _t4(c22_b2), _t4(c23_b2)])
    m1w1s = jnp.stack([_bd4(mlp1_W1[32 * k:32 * k + 32]) for k in range(3)])
    m2w1s = jnp.stack([_bd4(mlp2_W1[32 * k:32 * k + 32]) for k in range(3)])
    m1w2bd = _bd4(mlp1_W2)
    m2w2bd = _bd4(mlp2_W2)
    w1s2 = jnp.stack([_bd4(c21_W1), _bd4(c22_W1), _bd4(c23_W1)])

    x4 = x.reshape(n // 4, 512)
    y0p, y1p, y2p = _premul3(x4, bdw1s)
    ps = _segsum3(y0p.reshape(n, 32), y1p.reshape(n, 32), y2p.reshape(n, 32),
                  edge_index_0, edge_index_1, edge_index_2)
    ps4 = [p.reshape(p.shape[0] // 4, 128) for p in ps]
    z0p, z1p, z2p = _mid(y0p, y1p, y2p, ps4, b1s_1, w2s_1, b2s_1,
                         m1w1s, _t4(mlp1_b1).reshape(1, 128), m1w2bd,
                         _t4(mlp1_b2).reshape(1, 128), w1s2)

    qs = _segsum3(z0p.reshape(n, 32), z1p.reshape(n, 32), z2p.reshape(n, 32),
                  edge_index_0, edge_index_1, edge_index_2)
    qs4 = [q.reshape(q.shape[0] // 4, 128) for q in qs]
    batch4 = batch.reshape(n // 4, 4)
    out = _final(z0p, z1p, z2p, qs4, batch4, b1s_2, w2s_2, b2s_2,
                 m2w1s, _t4(mlp2_b1).reshape(1, 128), m2w2bd,
                 _t4(mlp2_b2).reshape(1, 128), lin_W, lin_b.reshape(1, 1), g)
    return jnp.squeeze(out, axis=-1)


# speedup vs baseline: 1.0966x; 1.0966x over previous
import functools

import jax
import jax.numpy as jnp
from jax import lax
from jax.experimental import pallas as pl
from jax.experimental.pallas import tpu as pltpu
from jax.experimental.pallas import tpu_sc as plsc

BLK = 2000


def _bd4(w):
    return jnp.kron(jnp.eye(4, dtype=jnp.float32), w)


def _t4(b):
    return jnp.tile(b, 4)


def _mm_kernel(x_ref, w0, w1, w2, o0, o1, o2):
    xb = x_ref[...]
    for w, o in ((w0, o0), (w1, o1), (w2, o2)):
        o[...] = jnp.dot(xb, w[...], preferred_element_type=jnp.float32)


def _premul3(x4, bdw1s):
    n4, d4 = x4.shape
    blk4 = n4
    grid = 1
    outs = [jax.ShapeDtypeStruct((n4, 128), jnp.float32)] * 3
    return pl.pallas_call(
        _mm_kernel,
        grid=(grid,),
        in_specs=[pl.BlockSpec((blk4, d4), lambda i: (i, 0))] + [
            pl.BlockSpec((d4, 128), lambda i: (0, 0))] * 3,
        out_specs=[pl.BlockSpec((blk4, 128), lambda i: (i, 0))] * 3,
        out_shape=outs,
    )(x4, *bdw1s)


def _segsum3(y0, y1, y2, e0, e1, e2):
    n = y0.shape[0]
    e = e0.shape[1]
    info = plsc.get_sparse_core_info()
    nc, ns = info.num_cores, info.num_subcores
    nw = nc * ns
    epw = e // nw
    ch = 200
    nch = epw // ch
    nb = 9
    assert ch * nch == epw and epw * nw == e
    rpt = (-(-n // ns) + 7) // 8 * 8
    n_pad = rpt * ns

    @functools.partial(
        pl.kernel,
        out_type=[jax.ShapeDtypeStruct((n_pad, 32), jnp.float32)] * 6,
        mesh=plsc.VectorSubcoreMesh(core_axis_name="c", subcore_axis_name="s"),
        scratch_types=[
            pltpu.VMEM_SHARED((n_pad, 32), jnp.float32),
            pltpu.VMEM_SHARED((n_pad, 32), jnp.float32),
            pltpu.VMEM_SHARED((n_pad, 32), jnp.float32),
            [pltpu.VMEM((ch,), jnp.int32) for _ in range(nb)],
            [pltpu.VMEM((ch,), jnp.int32) for _ in range(nb)],
            [pltpu.VMEM((ch, 32), jnp.float32) for _ in range(nb)],
            [pltpu.SemaphoreType.DMA for _ in range(nb)],
            [pltpu.SemaphoreType.DMA for _ in range(nb)],
            [pltpu.SemaphoreType.DMA for _ in range(nb)],
        ],
        compiler_params=pltpu.CompilerParams(use_tc_tiling_on_sc=False),
    )
    def k(y0h, y1h, y2h, e0h, e1h, e2h,
          o00, o01, o10, o11, o20, o21,
          a0, a1, a2, sidx, didx, rows, isem, gsem, ssem):
        cid = lax.axis_index("c")
        sid = lax.axis_index("s")
        wid = sid * nc + cid
        base = wid * epw
        rows_a = rows[0]

        def prefetch(eh, c, b):
            pltpu.async_copy(eh.at[0, pl.ds(base + c * ch, ch)],
                             sidx[b], isem[b])
            pltpu.async_copy(eh.at[1, pl.ds(base + c * ch, ch)],
                             didx[b], isem[b])

        def wait_prefetch(eh, c, b):
            pltpu.make_async_copy(eh.at[0, pl.ds(base + c * ch, ch)],
                                  sidx[b], isem[b]).wait()
            pltpu.make_async_copy(eh.at[1, pl.ds(base + c * ch, ch)],
                                  didx[b], isem[b]).wait()

        for b in range(nb):
            prefetch(e0h, b, b)

        zero16 = jnp.zeros((16,), jnp.float32)

        def zb(i, carry):
            rows_a[i, pl.ds(0, 16)] = zero16
            rows_a[i, pl.ds(16, 16)] = zero16
            return carry

        lax.fori_loop(0, ch, zb, 0)
        r0 = sid * rpt
        nfull, rem = divmod(rpt, ch)
        zdscs = []
        zi = 0
        for a in (a0, a1, a2):
            for j in range(nfull):
                zdscs.append(pltpu.async_copy(
                    rows_a, a.at[pl.ds(r0 + j * ch, ch)], gsem[zi % nb]))
                zi += 1
            if rem:
                zdscs.append(pltpu.async_copy(
                    rows_a.at[pl.ds(0, rem)],
                    a.at[pl.ds(r0 + nfull * ch, rem)], gsem[zi % nb]))
                zi += 1
        for dsc in zdscs:
            dsc.wait()
        plsc.subcore_barrier()

        ngrp = nch // nb
        npeel = nch - ngrp * nb
        sets = ((y0h, e0h, a0), (y1h, e1h, a1), (y2h, e2h, a2))
        for ksi, (yh, eh, a) in enumerate(sets):
            if ksi > 0:
                for b in range(nb):
                    prefetch(eh, b, b)

            def grp(g, carry):
                for b in range(nb):
                    c = g * nb + b
                    wait_prefetch(eh, c, b)
                    pltpu.async_copy(yh.at[sidx[b]], rows[b], gsem[b])
                for b in range(nb):
                    pltpu.make_async_copy(yh.at[sidx[b]], rows[b],
                                          gsem[b]).wait()
                    pltpu.async_copy(rows[b], a.at[didx[b]], ssem[b],
                                     add=True)
                for b in range(nb):
                    c_next = g * nb + b + nb

                    @pl.when(c_next < nch)
                    def _():
                        pltpu.make_async_copy(rows[b], a.at[didx[b]],
                                              ssem[b]).wait()
                        prefetch(eh, c_next, b)
                return carry

            lax.fori_loop(0, ngrp, grp, 0)
            for b in range(npeel):
                c = ngrp * nb + b
                wait_prefetch(eh, c, b)
                pltpu.async_copy(yh.at[sidx[b]], rows[b], gsem[b])
            for b in range(npeel):
                pltpu.make_async_copy(yh.at[sidx[b]], rows[b], gsem[b]).wait()
                pltpu.async_copy(rows[b], a.at[didx[b]], ssem[b], add=True)
            for b in range(nb):
                pltpu.make_async_copy(rows[b], a.at[didx[b]], ssem[b]).wait()
        plsc.subcore_barrier()

        for a, oc0, oc1 in ((a0, o00, o01), (a1, o10, o11), (a2, o20, o21)):
            @pl.when(cid == 0)
            def _():
                pltpu.sync_copy(a.at[pl.ds(r0, rpt)], oc0.at[pl.ds(r0, rpt)])

            @pl.when(cid == 1)
            def _():
                pltpu.sync_copy(a.at[pl.ds(r0, rpt)], oc1.at[pl.ds(r0, rpt)])

    return k(y0, y1, y2, e0, e1, e2)


def _mid_kernel(y0, y1, y2, p00, p01, p10, p11, p20, p21, b1s, w2s, b2s,
                m1w1s, m1b1, m1w2, m1b2, w1s2, o0, o1, o2):
    upre = m1b1[...]
    for i, (y, pa, pb) in enumerate(((y0, p00, p01), (y1, p10, p11),
                                     (y2, p20, p21))):
        nr = y.shape[0]
        pre = (y[...] + pa[pl.ds(0, nr), :] + pb[pl.ds(0, nr), :]
               + b1s[pl.ds(i, 1)])
        t = jnp.dot(jnp.maximum(pre, 0.0), w2s[i],
                    preferred_element_type=jnp.float32) + b2s[pl.ds(i, 1)]
        upre = upre + jnp.dot(t, m1w1s[i], preferred_element_type=jnp.float32)
    u = jnp.dot(jnp.maximum(upre, 0.0), m1w2[...],
                preferred_element_type=jnp.float32) + m1b2[...]
    for i, o in enumerate((o0, o1, o2)):
        o[...] = jnp.dot(u, w1s2[i], preferred_element_type=jnp.float32)


def _mid(y0, y1, y2, ps, b1s, w2s, b2s, m1w1s, m1b1, m1w2, m1b2, w1s2):
    n4 = y0.shape[0]
    blk4 = n4
    grid = 1
    yspec = pl.BlockSpec((blk4, 128), lambda i: (i, 0))
    pspec = pl.BlockSpec(ps[0].shape, lambda i: (0, 0))
    full = lambda s: pl.BlockSpec(s, lambda i: tuple(0 for _ in s))
    return pl.pallas_call(
        _mid_kernel,
        grid=(grid,),
        in_specs=[yspec] * 3 + [pspec] * 6 + [
            full((3, 128)), full((3, 128, 128)), full((3, 128)),
            full((3, 128, 128)), full((1, 128)), full((128, 128)),
            full((1, 128)), full((3, 128, 128)),
        ],
        out_specs=[yspec] * 3,
        out_shape=[jax.ShapeDtypeStruct((n4, 128), jnp.float32)] * 3,
    )(y0, y1, y2, *ps, b1s, w2s, b2s, m1w1s, m1b1, m1w2, m1b2, w1s2)


def _fin_kernel(z0, z1, z2, q00, q01, q10, q11, q20, q21, batch_ref,
                b1s, w2s, b2s, m2w1s, m2b1, m2w2, m2b2, linw, linb,
                out_ref, acc):
    i = pl.program_id(0)
    nblk = pl.num_programs(0)
    vpre = m2b1[...]
    for k, (z, qa, qb) in enumerate(((z0, q00, q01), (z1, q10, q11),
                                     (z2, q20, q21))):
        nr = z.shape[0]
        pre = (z[...] + qa[pl.ds(0, nr), :] + qb[pl.ds(0, nr), :]
               + b1s[pl.ds(k, 1)])
        t = jnp.dot(jnp.maximum(pre, 0.0), w2s[k],
                    preferred_element_type=jnp.float32) + b2s[pl.ds(k, 1)]
        vpre = vpre + jnp.dot(jnp.maximum(t, 0.0), m2w1s[k],
                              preferred_element_type=jnp.float32)
    v = jnp.dot(jnp.maximum(vpre, 0.0), m2w2[...],
                preferred_element_type=jnp.float32) + m2b2[...]
    g = acc.shape[0]
    blk4 = v.shape[0]
    part = jnp.zeros((g, 32), jnp.float32)
    for j in range(4):
        bj = batch_ref[:, pl.ds(j, 1)]
        ohjt = jnp.where(
            jax.lax.broadcasted_iota(jnp.int32, (blk4, g), 1) == bj, 1.0, 0.0)
        vj = v[:, 32 * j:32 * j + 32]
        part = part + jax.lax.dot_general(
            ohjt, vj, (((0,), (0,)), ((), ())),
            preferred_element_type=jnp.float32)

    @pl.when(i == 0)
    def _():
        acc[...] = jnp.zeros_like(acc)

    acc[...] += part

    @pl.when(i == nblk - 1)
    def _():
        out_ref[...] = jnp.dot(acc[...], linw[...],
                               preferred_element_type=jnp.float32) + linb[...]


def _final(z0, z1, z2, qs, batch4, b1s, w2s, b2s,
           m2w1s, m2b1, m2w2, m2b2, linw, linb, g):
    n4 = z0.shape[0]
    blk4 = n4
    grid = 1
    zspec = pl.BlockSpec((blk4, 128), lambda i: (i, 0))
    qspec = pl.BlockSpec(qs[0].shape, lambda i: (0, 0))
    full = lambda s: pl.BlockSpec(s, lambda i: tuple(0 for _ in s))
    return pl.pallas_call(
        _fin_kernel,
        grid=(grid,),
        in_specs=[zspec] * 3 + [qspec] * 6 + [
            pl.BlockSpec((blk4, 4), lambda i: (i, 0)),
            full((3, 128)), full((3, 128, 128)), full((3, 128)),
            full((3, 128, 128)), full((1, 128)), full((128, 128)),
            full((1, 128)), full((32, 1)), full((1, 1)),
        ],
        out_specs=full((g, 1)),
        out_shape=jax.ShapeDtypeStruct((g, 1), jnp.float32),
        scratch_shapes=[pltpu.VMEM((g, 32), jnp.float32)],
    )(z0, z1, z2, *qs, batch4, b1s, w2s, b2s,
      m2w1s, m2b1, m2w2, m2b2, linw, linb)


def kernel(x, edge_index_0, edge_index_1, edge_index_2, batch,
           c11_W1, c11_b1, c11_W2, c11_b2,
           c12_W1, c12_b1, c12_W2, c12_b2,
           c13_W1, c13_b1, c13_W2, c13_b2,
           c21_W1, c21_b1, c21_W2, c21_b2,
           c22_W1, c22_b1, c22_W2, c22_b2,
           c23_W1, c23_b1, c23_W2, c23_b2,
           mlp1_W1, mlp1_b1, mlp1_W2, mlp1_b2,
           mlp2_W1, mlp2_b1, mlp2_W2, mlp2_b2,
           lin_W, lin_b):
    n = x.shape[0]
    g = 64

    bdw1s = [jnp.kron(jnp.eye(4, dtype=jnp.float32), w)
             for w in (c11_W1, c12_W1, c13_W1)]
    b1s_1 = jnp.stack([_t4(c11_b1), _t4(c12_b1), _t4(c13_b1)])
    w2s_1 = jnp.stack([_bd4(c11_W2), _bd4(c12_W2), _bd4(c13_W2)])
    b2s_1 = jnp.stack([_t4(c11_b2), _t4(c12_b2), _t4(c13_b2)])
    b1s_2 = jnp.stack([_t4(c21_b1), _t4(c22_b1), _t4(c23_b1)])
    w2s_2 = jnp.stack([_bd4(c21_W2), _bd4(c22_W2), _bd4(c23_W2)])
    b2s_2 = jnp.stack([_t4(c21_b2), _t4(c22_b2), _t4(c23_b2)])
    m1w1s = jnp.stack([_bd4(mlp1_W1[32 * k:32 * k + 32]) for k in range(3)])
    m2w1s = jnp.stack([_bd4(mlp2_W1[32 * k:32 * k + 32]) for k in range(3)])
    m1w2bd = _bd4(mlp1_W2)
    m2w2bd = _bd4(mlp2_W2)
    w1s2 = jnp.stack([_bd4(c21_W1), _bd4(c22_W1), _bd4(c23_W1)])

    x4 = x.reshape(n // 4, 512)
    y0p, y1p, y2p = _premul3(x4, bdw1s)
    ps = _segsum3(y0p.reshape(n, 32), y1p.reshape(n, 32), y2p.reshape(n, 32),
                  edge_index_0, edge_index_1, edge_index_2)
    ps4 = [p.reshape(p.shape[0] // 4, 128) for p in ps]
    z0p, z1p, z2p = _mid(y0p, y1p, y2p, ps4, b1s_1, w2s_1, b2s_1,
                         m1w1s, _t4(mlp1_b1).reshape(1, 128), m1w2bd,
                         _t4(mlp1_b2).reshape(1, 128), w1s2)

    qs = _segsum3(z0p.reshape(n, 32), z1p.reshape(n, 32), z2p.reshape(n, 32),
                  edge_index_0, edge_index_1, edge_index_2)
    qs4 = [q.reshape(q.shape[0] // 4, 128) for q in qs]
    batch4 = batch.reshape(n // 4, 4)
    out = _final(z0p, z1p, z2p, qs4, batch4, b1s_2, w2s_2, b2s_2,
                 m2w1s, _t4(mlp2_b1).reshape(1, 128), m2w2bd,
                 _t4(mlp2_b2).reshape(1, 128), lin_W, lin_b.reshape(1, 1), g)
    return jnp.squeeze(out, axis=-1)
